# baseline (speedup 1.0000x reference)
import jax
import jax.numpy as jnp
from jax import lax
from jax.experimental import pallas as pl
from jax.experimental.pallas import tpu as pltpu
from jax.experimental.pallas import tpu_sc as plsc

NC = 2
NS = 16
CH = 40
NEG = -1e30


def _node_kernel(x_ref, w1_ref, a_ref, psx_ref, pt_ref, t2_ref, mx_ref):
    x = x_ref[...]
    w1 = w1_ref[...]
    a = a_ref[...]
    ws = w1[:, :128]
    wt = w1[:, 128:256]
    dn = (((1,), (1,)), ((), ()))
    ps = lax.dot_general(x, ws, dn, preferred_element_type=jnp.float32)
    pt = lax.dot_general(x, wt, dn, preferred_element_type=jnp.float32)
    da = (((1,), (0,)), ((), ()))
    ss = lax.dot_general(ps, a, da, preferred_element_type=jnp.float32)
    st = lax.dot_general(pt, a, da, preferred_element_type=jnp.float32)
    psx_ref[...] = jnp.concatenate([ps, ss, ss], axis=1)
    pt_ref[...] = pt
    t2_ref[...] = jnp.concatenate([st, st], axis=1)
    pad = jnp.full((1, 112), NEG, jnp.float32)
    row = jnp.concatenate([jnp.max(ss, axis=0, keepdims=True),
                           jnp.max(st, axis=0, keepdims=True), pad], axis=1)
    mx_ref[...] = row.reshape(1, 1, 128)


def _edge_kernel(r_ref, w1r_ref, rw_ref, a_ref, c_ref, mx_ref):
    r = r_ref[...]
    w1r = w1r_ref[...]
    rw = rw_ref[...]
    a = a_ref[...]
    wc = lax.dot_general(w1r, rw, (((1,), (0,)), ((), ())),
                         preferred_element_type=jnp.float32)
    c = lax.dot_general(r, wc, (((1,), (1,)), ((), ())),
                        preferred_element_type=jnp.float32)
    sr = lax.dot_general(c, a, (((1,), (0,)), ((), ())),
                         preferred_element_type=jnp.float32)
    c_ref[...] = jnp.concatenate([c, sr, sr], axis=1)
    pad = jnp.full((1, 120), NEG, jnp.float32)
    row = jnp.concatenate([jnp.max(sr, axis=0, keepdims=True), pad], axis=1)
    mx_ref[...] = row.reshape(1, 1, 128)


def _sc_kernel(src_hbm, tgt_hbm, psx_hbm, t2_hbm, cpk_hbm, mv_hbm,
               zrow_hbm, u_hbm,
               idxs0, idxs1, idxt0, idxt1,
               t2b0, t2b1, psxb0, psxb1, cpkb0, cpkb1, ybuf0, ybuf1, mv_v,
               u_sh,
               semL1_0, semL1_1, semL2_0, semL2_1):
    cid = lax.axis_index("c")
    sid = lax.axis_index("s")
    n = psx_hbm.shape[0]
    e_total = src_hbm.shape[0]
    e_sc = e_total // NC
    nch_sc = e_sc // CH
    ncht = nch_sc // NS
    npairs = ncht // 2
    idxs = (idxs0, idxs1)
    idxt = (idxt0, idxt1)
    t2b = (t2b0, t2b1)
    psxb = (psxb0, psxb1)
    cpkb = (cpkb0, cpkb1)
    ybuf = (ybuf0, ybuf1)
    semL1 = (semL1_0, semL1_1)
    semL2 = (semL2_0, semL2_1)
    sc_base = cid * e_sc

    nr = (n // NS) // 8 * 8
    r0 = sid * nr
    t0 = nr * NS
    nt = n - t0

    pltpu.sync_copy(zrow_hbm.at[pl.ds(r0, nr)], u_sh.at[pl.ds(r0, nr)])

    @pl.when(sid == NS - 1)
    def _init_tail():
        pltpu.sync_copy(zrow_hbm.at[pl.ds(t0, nt)], u_sh.at[pl.ds(t0, nt)])

    pltpu.sync_copy(mv_hbm, mv_v)
    plsc.subcore_barrier()

    def issue_l1(b, j):
        base = sc_base + (j * NS + sid) * CH
        pltpu.async_copy(src_hbm.at[pl.ds(base, CH)], idxs[b], semL1[b])
        pltpu.async_copy(tgt_hbm.at[pl.ds(base, CH)], idxt[b], semL1[b])
        pltpu.async_copy(cpk_hbm.at[pl.ds(base, CH)], cpkb[b], semL1[b])

    def wait_l1(b):
        pltpu.make_async_copy(src_hbm.at[pl.ds(0, CH)], idxs[b], semL1[b]).wait()
        pltpu.make_async_copy(tgt_hbm.at[pl.ds(0, CH)], idxt[b], semL1[b]).wait()
        pltpu.make_async_copy(cpk_hbm.at[pl.ds(0, CH)], cpkb[b], semL1[b]).wait()

    def issue_l2(b):
        pltpu.async_copy(psx_hbm.at[idxs[b]], psxb[b], semL2[b])
        pltpu.async_copy(t2_hbm.at[idxt[b]], t2b[b], semL2[b])

    def wait_l2(b):
        pltpu.make_async_copy(psx_hbm.at[idxs[b]], psxb[b], semL2[b]).wait()
        pltpu.make_async_copy(t2_hbm.at[idxt[b]], t2b[b], semL2[b]).wait()

    def compute(b):
        mreg = mv_v[...]

        def edge_body(e, c2):
            sc = psxb[b][e, pl.ds(128, 16)] + t2b[b][e, :] + cpkb[b][e, pl.ds(128, 16)]
            sc = jnp.where(sc > 0, sc, 0.2 * sc)
            ex = jnp.exp(sc - mreg)
            ybuf[b][e, pl.ds(128, 16)] = ex
            for h in range(8):
                eh = jnp.full((16,), ex[h], jnp.float32)
                blk = psxb[b][e, pl.ds(h * 16, 16)] + cpkb[b][e, pl.ds(h * 16, 16)]
                ybuf[b][e, pl.ds(h * 16, 16)] = blk * eh
            return c2

        lax.fori_loop(0, CH, edge_body, 0, unroll=4)

    def issue_sc(b):
        pltpu.sync_copy(ybuf[b], u_sh.at[idxt[b]], add=True)

    issue_l1(0, 0)
    issue_l1(1, 1)
    wait_l1(0)
    issue_l2(0)

    def pair_body(k, carry):
        wait_l2(0)

        wait_l1(1)
        issue_l2(1)

        compute(0)
        issue_sc(0)

        @pl.when(k < npairs - 1)
        def _():
            issue_l1(0, 2 * k + 2)

        wait_l2(1)

        @pl.when(k < npairs - 1)
        def _():
            wait_l1(0)
            issue_l2(0)

        compute(1)
        issue_sc(1)

        @pl.when(k < npairs - 1)
        def _():
            issue_l1(1, 2 * k + 3)

        return carry

    lax.fori_loop(0, npairs, pair_body, 0)

    nrem = nch_sc - ncht * NS

    @pl.when(sid < nrem)
    def _remainder():
        issue_l1(0, ncht)
        wait_l1(0)
        issue_l2(0)
        wait_l2(0)
        compute(0)
        issue_sc(0)

    plsc.subcore_barrier()
    pltpu.sync_copy(u_sh.at[pl.ds(r0, nr)], u_hbm.at[cid, pl.ds(r0, nr)])

    @pl.when(sid == NS - 1)
    def _export_tail():
        pltpu.sync_copy(u_sh.at[pl.ds(t0, nt)], u_hbm.at[cid, pl.ds(t0, nt)])


def _combine_kernel(u_ref, pt_ref, b_ref, o_ref):
    blk = u_ref[0] + u_ref[1]
    u = blk[:, :128]
    srep = jnp.repeat(blk[:, 128:136], 16, axis=1)
    o_ref[...] = (u + srep * pt_ref[...]) / (srep + 1e-16) + b_ref[0][None, :]


def kernel(inp, relation_inp, dep_rel_pos_edge, word_rel_pos_edge, deprel_edge,
           deprel_ext_edge, deparc_edge, edge_index, dist_edge, deprel_path_edge,
           deparc_path_edge, path_len_edge, deprel_ext_path_edge,
           W1, weight2, rel_weight, final_bias):
    f32 = jnp.float32
    n, d = inp.shape
    e, r = relation_inp.shape
    h, dv, _ = weight2.shape
    w = d + 16

    w2v = weight2[..., 0]
    A = (w2v[:, :, None] * jnp.eye(h, dtype=f32)[:, None, :]).reshape(h * dv, h)

    bn = 1000
    ga = n // bn
    PSX, PT, T2, MXA = pl.pallas_call(
        _node_kernel,
        grid=(ga,),
        in_specs=[pl.BlockSpec((bn, d), lambda i: (i, 0)),
                  pl.BlockSpec((h * dv, 2 * d + r), lambda i: (0, 0)),
                  pl.BlockSpec((h * dv, h), lambda i: (0, 0))],
        out_specs=[pl.BlockSpec((bn, w), lambda i: (i, 0)),
                   pl.BlockSpec((bn, d), lambda i: (i, 0)),
                   pl.BlockSpec((bn, 16), lambda i: (i, 0)),
                   pl.BlockSpec((1, 1, 128), lambda i: (i, 0, 0))],
        out_shape=[jax.ShapeDtypeStruct((n, w), f32),
                   jax.ShapeDtypeStruct((n, d), f32),
                   jax.ShapeDtypeStruct((n, 16), f32),
                   jax.ShapeDtypeStruct((ga, 1, 128), f32)],
    )(inp, W1, A)

    be = 8000
    gb = e // be
    CPK, MXB = pl.pallas_call(
        _edge_kernel,
        grid=(gb,),
        in_specs=[pl.BlockSpec((be, r), lambda i: (i, 0)),
                  pl.BlockSpec((h * dv, r), lambda i: (0, 0)),
                  pl.BlockSpec((r, r), lambda i: (0, 0)),
                  pl.BlockSpec((h * dv, h), lambda i: (0, 0))],
        out_specs=[pl.BlockSpec((be, w), lambda i: (i, 0)),
                   pl.BlockSpec((1, 1, 128), lambda i: (i, 0, 0))],
        out_shape=[jax.ShapeDtypeStruct((e, w), f32),
                   jax.ShapeDtypeStruct((gb, 1, 128), f32)],
    )(relation_inp, W1[:, 2 * d:], rel_weight, A)

    mA = jnp.max(MXA[:, 0, :], axis=0)
    mB = jnp.max(MXB[:, 0, :], axis=0)
    m_raw = jnp.max(mA[:8] + mA[8:16] + mB[:8])
    m = jnp.where(m_raw > 0, m_raw, 0.2 * m_raw)
    mvec = jnp.full((16,), m, f32)

    mesh = plsc.VectorSubcoreMesh(core_axis_name="c", subcore_axis_name="s",
                                  num_cores=NC, num_subcores=NS)
    scatter = pl.kernel(
        _sc_kernel,
        out_type=jax.ShapeDtypeStruct((NC, n, w), f32),
        mesh=mesh,
        compiler_params=pltpu.CompilerParams(use_tc_tiling_on_sc=False),
        scratch_types=(
            [pltpu.VMEM((CH,), jnp.int32)] * 4 +
            [pltpu.VMEM((CH, 16), f32)] * 2 +
            [pltpu.VMEM((CH, w), f32)] * 6 +
            [pltpu.VMEM((16,), f32),
             pltpu.VMEM_SHARED((n, w), f32)] +
            [pltpu.SemaphoreType.DMA] * 4
        ),
    )
    U = scatter(edge_index[0], edge_index[1], PSX, T2, CPK, mvec,
                jnp.zeros((n, w), f32))

    bias2 = jnp.broadcast_to(final_bias, (8, d))
    out = pl.pallas_call(
        _combine_kernel,
        grid=(ga,),
        in_specs=[pl.BlockSpec((NC, bn, w), lambda i: (0, i, 0)),
                  pl.BlockSpec((bn, d), lambda i: (i, 0)),
                  pl.BlockSpec((8, d), lambda i: (0, 0))],
        out_specs=pl.BlockSpec((bn, d), lambda i: (i, 0)),
        out_shape=jax.ShapeDtypeStruct((n, d), f32),
    )(U, PT, bias2)
    return out


# speedup vs baseline: 1.3664x; 1.3664x over previous
import jax
import jax.numpy as jnp
from jax import lax
from jax.experimental import pallas as pl
from jax.experimental.pallas import tpu as pltpu
from jax.experimental.pallas import tpu_sc as plsc

NC = 2
NS = 16
CH = 40
NEG = -1e30


def _node_kernel(x_ref, w1_ref, a_ref, psx_ref, pt_ref, t2_ref, mx_ref):
    x = x_ref[...]
    w1 = w1_ref[...]
    a = a_ref[...]
    ws = w1[:, :128]
    wt = w1[:, 128:256]
    dn = (((1,), (1,)), ((), ()))
    ps = lax.dot_general(x, ws, dn, preferred_element_type=jnp.float32)
    pt = lax.dot_general(x, wt, dn, preferred_element_type=jnp.float32)
    da = (((1,), (0,)), ((), ()))
    ss = lax.dot_general(ps, a, da, preferred_element_type=jnp.float32)
    st = lax.dot_general(pt, a, da, preferred_element_type=jnp.float32)
    psx_ref[...] = jnp.concatenate([ps, ss, ss], axis=1)
    pt_ref[...] = pt
    t2_ref[...] = jnp.concatenate([st, st], axis=1)
    pad = jnp.full((1, 112), NEG, jnp.float32)
    row = jnp.concatenate([jnp.max(ss, axis=0, keepdims=True),
                           jnp.max(st, axis=0, keepdims=True), pad], axis=1)
    mx_ref[...] = row.reshape(1, 1, 128)


def _edge_kernel(r_ref, w1r_ref, rw_ref, a_ref, c_ref, mx_ref):
    r = r_ref[...]
    w1r = w1r_ref[...]
    rw = rw_ref[...]
    a = a_ref[...]
    wc = lax.dot_general(w1r, rw, (((1,), (0,)), ((), ())),
                         preferred_element_type=jnp.float32)
    c = lax.dot_general(r, wc, (((1,), (1,)), ((), ())),
                        preferred_element_type=jnp.float32)
    sr = lax.dot_general(c, a, (((1,), (0,)), ((), ())),
                         preferred_element_type=jnp.float32)
    c_ref[...] = jnp.concatenate([c, sr, sr], axis=1)
    pad = jnp.full((1, 120), NEG, jnp.float32)
    row = jnp.concatenate([jnp.max(sr, axis=0, keepdims=True), pad], axis=1)
    mx_ref[...] = row.reshape(1, 1, 128)


def _sc_kernel(src_hbm, tgt_hbm, psx_hbm, t2_hbm, cpk_hbm, mv_hbm,
               zrow_hbm, u_hbm,
               idxs0, idxs1, idxt0, idxt1,
               t2b0, t2b1, psxb0, psxb1, cpkb0, cpkb1, ybuf0, ybuf1, mv_v,
               u_sh,
               semL1_0, semL1_1, semL2_0, semL2_1):
    cid = lax.axis_index("c")
    sid = lax.axis_index("s")
    n = psx_hbm.shape[0]
    e_total = src_hbm.shape[0]
    e_sc = e_total // NC
    nch_sc = e_sc // CH
    ncht = nch_sc // NS
    npairs = ncht // 2
    idxs = (idxs0, idxs1)
    idxt = (idxt0, idxt1)
    t2b = (t2b0, t2b1)
    psxb = (psxb0, psxb1)
    cpkb = (cpkb0, cpkb1)
    ybuf = (ybuf0, ybuf1)
    semL1 = (semL1_0, semL1_1)
    semL2 = (semL2_0, semL2_1)
    sc_base = cid * e_sc

    nr = (n // NS) // 8 * 8
    r0 = sid * nr
    t0 = nr * NS
    nt = n - t0

    pltpu.sync_copy(zrow_hbm.at[pl.ds(r0, nr)], u_sh.at[pl.ds(r0, nr)])

    @pl.when(sid == NS - 1)
    def _init_tail():
        pltpu.sync_copy(zrow_hbm.at[pl.ds(t0, nt)], u_sh.at[pl.ds(t0, nt)])

    pltpu.sync_copy(mv_hbm, mv_v)
    plsc.subcore_barrier()

    def issue_l1(b, j):
        base = sc_base + (j * NS + sid) * CH
        pltpu.async_copy(src_hbm.at[pl.ds(base, CH)], idxs[b], semL1[b])
        pltpu.async_copy(tgt_hbm.at[pl.ds(base, CH)], idxt[b], semL1[b])
        pltpu.async_copy(cpk_hbm.at[pl.ds(base, CH)], cpkb[b], semL1[b])

    def wait_l1(b):
        pltpu.make_async_copy(src_hbm.at[pl.ds(0, CH)], idxs[b], semL1[b]).wait()
        pltpu.make_async_copy(tgt_hbm.at[pl.ds(0, CH)], idxt[b], semL1[b]).wait()
        pltpu.make_async_copy(cpk_hbm.at[pl.ds(0, CH)], cpkb[b], semL1[b]).wait()

    def issue_l2(b):
        pltpu.async_copy(psx_hbm.at[idxs[b]], psxb[b], semL2[b])
        pltpu.async_copy(t2_hbm.at[idxt[b]], t2b[b], semL2[b])

    def wait_l2(b):
        pltpu.make_async_copy(psx_hbm.at[idxs[b]], psxb[b], semL2[b]).wait()
        pltpu.make_async_copy(t2_hbm.at[idxt[b]], t2b[b], semL2[b]).wait()

    def compute(b):
        mreg = mv_v[...]

        def edge_body(e, c2):
            sc = psxb[b][e, pl.ds(128, 16)] + t2b[b][e, :] + cpkb[b][e, pl.ds(128, 16)]
            sc = jnp.where(sc > 0, sc, 0.2 * sc)
            ex = jnp.exp(sc - mreg)
            ybuf[b][e, pl.ds(128, 16)] = ex
            for h in range(8):
                eh = jnp.full((16,), ex[h], jnp.float32)
                blk = psxb[b][e, pl.ds(h * 16, 16)] + cpkb[b][e, pl.ds(h * 16, 16)]
                ybuf[b][e, pl.ds(h * 16, 16)] = blk * eh
            return c2

        lax.fori_loop(0, CH, edge_body, 0)

    def issue_sc(b):
        pltpu.sync_copy(ybuf[b], u_sh.at[idxt[b]], add=True)

    issue_l1(0, 0)
    issue_l1(1, 1)
    wait_l1(0)
    issue_l2(0)

    def pair_body(k, carry):
        wait_l2(0)

        wait_l1(1)
        issue_l2(1)

        compute(0)
        issue_sc(0)

        @pl.when(k < npairs - 1)
        def _():
            issue_l1(0, 2 * k + 2)

        wait_l2(1)

        @pl.when(k < npairs - 1)
        def _():
            wait_l1(0)
            issue_l2(0)

        compute(1)
        issue_sc(1)

        @pl.when(k < npairs - 1)
        def _():
            issue_l1(1, 2 * k + 3)

        return carry

    lax.fori_loop(0, npairs, pair_body, 0)

    nrem = nch_sc - ncht * NS

    @pl.when(sid < nrem)
    def _remainder():
        issue_l1(0, ncht)
        wait_l1(0)
        issue_l2(0)
        wait_l2(0)
        compute(0)
        issue_sc(0)

    plsc.subcore_barrier()
    pltpu.sync_copy(u_sh.at[pl.ds(r0, nr)], u_hbm.at[cid, pl.ds(r0, nr)])

    @pl.when(sid == NS - 1)
    def _export_tail():
        pltpu.sync_copy(u_sh.at[pl.ds(t0, nt)], u_hbm.at[cid, pl.ds(t0, nt)])


def _combine_kernel(u_ref, pt_ref, b_ref, o_ref):
    blk = u_ref[0] + u_ref[1]
    u = blk[:, :128]
    srep = jnp.repeat(blk[:, 128:136], 16, axis=1)
    o_ref[...] = (u + srep * pt_ref[...]) / (srep + 1e-16) + b_ref[0][None, :]


def kernel(inp, relation_inp, dep_rel_pos_edge, word_rel_pos_edge, deprel_edge,
           deprel_ext_edge, deparc_edge, edge_index, dist_edge, deprel_path_edge,
           deparc_path_edge, path_len_edge, deprel_ext_path_edge,
           W1, weight2, rel_weight, final_bias):
    f32 = jnp.float32
    n, d = inp.shape
    e, r = relation_inp.shape
    h, dv, _ = weight2.shape
    w = d + 16

    w2v = weight2[..., 0]
    A = (w2v[:, :, None] * jnp.eye(h, dtype=f32)[:, None, :]).reshape(h * dv, h)

    bn = 1000
    ga = n // bn
    PSX, PT, T2, MXA = pl.pallas_call(
        _node_kernel,
        grid=(ga,),
        in_specs=[pl.BlockSpec((bn, d), lambda i: (i, 0)),
                  pl.BlockSpec((h * dv, 2 * d + r), lambda i: (0, 0)),
                  pl.BlockSpec((h * dv, h), lambda i: (0, 0))],
        out_specs=[pl.BlockSpec((bn, w), lambda i: (i, 0)),
                   pl.BlockSpec((bn, d), lambda i: (i, 0)),
                   pl.BlockSpec((bn, 16), lambda i: (i, 0)),
                   pl.BlockSpec((1, 1, 128), lambda i: (i, 0, 0))],
        out_shape=[jax.ShapeDtypeStruct((n, w), f32),
                   jax.ShapeDtypeStruct((n, d), f32),
                   jax.ShapeDtypeStruct((n, 16), f32),
                   jax.ShapeDtypeStruct((ga, 1, 128), f32)],
    )(inp, W1, A)

    be = 8000
    gb = e // be
    CPK, MXB = pl.pallas_call(
        _edge_kernel,
        grid=(gb,),
        in_specs=[pl.BlockSpec((be, r), lambda i: (i, 0)),
                  pl.BlockSpec((h * dv, r), lambda i: (0, 0)),
                  pl.BlockSpec((r, r), lambda i: (0, 0)),
                  pl.BlockSpec((h * dv, h), lambda i: (0, 0))],
        out_specs=[pl.BlockSpec((be, w), lambda i: (i, 0)),
                   pl.BlockSpec((1, 1, 128), lambda i: (i, 0, 0))],
        out_shape=[jax.ShapeDtypeStruct((e, w), f32),
                   jax.ShapeDtypeStruct((gb, 1, 128), f32)],
    )(relation_inp, W1[:, 2 * d:], rel_weight, A)

    mA = jnp.max(MXA[:, 0, :], axis=0)
    mB = jnp.max(MXB[:, 0, :], axis=0)
    m_raw = jnp.max(mA[:8] + mA[8:16] + mB[:8])
    m = jnp.where(m_raw > 0, m_raw, 0.2 * m_raw)
    mvec = jnp.full((16,), m, f32)

    mesh = plsc.VectorSubcoreMesh(core_axis_name="c", subcore_axis_name="s",
                                  num_cores=NC, num_subcores=NS)
    scatter = pl.kernel(
        _sc_kernel,
        out_type=jax.ShapeDtypeStruct((NC, n, w), f32),
        mesh=mesh,
        compiler_params=pltpu.CompilerParams(use_tc_tiling_on_sc=False),
        scratch_types=(
            [pltpu.VMEM((CH,), jnp.int32)] * 4 +
            [pltpu.VMEM((CH, 16), f32)] * 2 +
            [pltpu.VMEM((CH, w), f32)] * 6 +
            [pltpu.VMEM((16,), f32),
             pltpu.VMEM_SHARED((n, w), f32)] +
            [pltpu.SemaphoreType.DMA] * 4
        ),
    )
    U = scatter(edge_index[0], edge_index[1], PSX, T2, CPK, mvec,
                jnp.zeros((n, w), f32))

    bias2 = jnp.broadcast_to(final_bias, (8, d))
    out = pl.pallas_call(
        _combine_kernel,
        grid=(ga,),
        in_specs=[pl.BlockSpec((NC, bn, w), lambda i: (0, i, 0)),
                  pl.BlockSpec((bn, d), lambda i: (i, 0)),
                  pl.BlockSpec((8, d), lambda i: (0, 0))],
        out_specs=pl.BlockSpec((bn, d), lambda i: (i, 0)),
        out_shape=jax.ShapeDtypeStruct((n, d), f32),
    )(U, PT, bias2)
    return out
